# baseline (speedup 1.0000x reference)
import jax
import jax.numpy as jnp
from jax.experimental import pallas as pl
from jax.experimental.pallas import tpu as pltpu
from jax.experimental.pallas import tpu_sc as plsc

_H = 256
_WINDOW = 128
_BLK = 128


def _gather_rows(emb, idx):
    n = idx.shape[0]
    d = emb.shape[1]
    idx2 = idx.reshape(1, n)

    @pl.kernel(
        out_type=jax.ShapeDtypeStruct((n, d), emb.dtype),
        mesh=plsc.VectorSubcoreMesh(core_axis_name="core",
                                    subcore_axis_name="subcore"),
    )
    def gather_kernel(x_hbm, i_hbm, o_hbm):
        def body(i_vmem, o_vmem):
            pltpu.sync_copy(x_hbm.at[i_vmem.at[0]], o_vmem)

        pltpu.emit_pipeline(
            body,
            grid=(n // _WINDOW,),
            in_specs=[pl.BlockSpec((1, _WINDOW), index_map=lambda i: (0, i))],
            out_specs=[pl.BlockSpec((_WINDOW, d), index_map=lambda i: (i, 0))],
            core_axis_name=("core", "subcore"),
            dimension_semantics=(pltpu.PARALLEL,),
        )(i_hbm, o_hbm)

    return gather_kernel(emb, idx2)


def _dense_body(xs_ref, lab_ref, xid_ref,
                Wif_ref, Whf_ref, bif_ref, bhf_ref,
                Wib_ref, Whb_ref, bib_ref, bhb_ref,
                ctxW_ref, ctxb_ref, inpW_ref, inpb_ref,
                headW_ref, headb_ref,
                head_ref):
    T, BLK, D = xs_ref.shape
    H = _H
    f32 = jnp.float32

    def dn(a, b):
        return jax.lax.dot_general(a, b, (((1,), (1,)), ((), ())),
                                   preferred_element_type=f32)

    xe = xs_ref[...].reshape(T * BLK, D)
    gi_f = (dn(xe, Wif_ref[...]) + bif_ref[...]).reshape(T, BLK, 3 * H)
    gi_b = (dn(xe, Wib_ref[...]) + bib_ref[...]).reshape(T, BLK, 3 * H)

    m = (xid_ref[...] != 0).astype(f32)

    Whf = Whf_ref[...]
    Whb = Whb_ref[...]
    bhf = bhf_ref[...]
    bhb = bhb_ref[...]

    def cell(h, gi_t, Wh, bh):
        gh = dn(h, Wh) + bh
        r = jax.nn.sigmoid(gi_t[:, :H] + gh[:, :H])
        z = jax.nn.sigmoid(gi_t[:, H:2 * H] + gh[:, H:2 * H])
        n = jnp.tanh(gi_t[:, 2 * H:] + r * gh[:, 2 * H:])
        return (1.0 - z) * n + z * h

    h_f = jnp.zeros((BLK, H), f32)
    h_b = jnp.zeros((BLK, H), f32)
    acc_f = jnp.zeros((BLK, H), f32)
    acc_b = jnp.zeros((BLK, H), f32)
    for t in range(T):
        tb = T - 1 - t
        h_f = cell(h_f, gi_f[t], Whf, bhf)
        acc_f = acc_f + m[:, t:t + 1] * h_f
        h_b = cell(h_b, gi_b[tb], Whb, bhb)
        acc_b = acc_b + m[:, tb:tb + 1] * h_b

    cnt = jnp.sum(m, axis=1, keepdims=True)
    cf = acc_f / cnt
    cb = acc_b / cnt

    ctxW = ctxW_ref[...]
    feat = (dn(lab_ref[...], inpW_ref[...]) + inpb_ref[...]
            + dn(cf, ctxW[:, :H]) + dn(cb, ctxW[:, H:]) + ctxb_ref[...])

    hv = dn(feat, headW_ref[...]) + headb_ref[...]
    lane = jax.lax.broadcasted_iota(jnp.int32, hv.shape, 1)
    head_ref[...] = jnp.where(lane < 2, jax.nn.sigmoid(hv), hv)


def kernel(x, label, emb, Wi_f, Wh_f, bi_f, bh_f, Wi_b, Wh_b, bi_b, bh_b,
           ctx_W, ctx_b, inp_W, inp_b, att_W, att_b, crit_W, crit_b):
    B, T = x.shape
    D = emb.shape[1]
    H = _H
    A = att_W.shape[0]
    f32 = jnp.float32

    x = x.astype(jnp.int32)
    label = label.astype(jnp.int32)

    idx = jnp.concatenate([x.T.reshape(-1), label.reshape(-1)])
    idx2 = jnp.stack([2 * idx, 2 * idx + 1], axis=1).reshape(-1)
    rows = _gather_rows(emb.reshape(2 * emb.shape[0], D // 2), idx2)
    rows = rows.reshape(-1, D)
    xs = rows[:B * T].reshape(T, B, D)
    lab = rows[B * T:]

    wd = att_W[0:1] - att_W[1:2]
    headW = jnp.concatenate([wd, -wd, crit_W,
                             jnp.zeros((128 - 3, H), f32)], axis=0)
    bd = att_b[0] - att_b[1]
    headb = jnp.concatenate([bd[None], -bd[None], crit_b,
                             jnp.zeros((128 - 3,), f32)]).reshape(1, 128)

    def full(shape):
        return pl.BlockSpec(shape, lambda *_: tuple(0 for _ in shape))

    grid = (B // _BLK,)
    head = pl.pallas_call(
        _dense_body,
        grid=grid,
        in_specs=[
            pl.BlockSpec((T, _BLK, D), lambda i: (0, i, 0)),
            pl.BlockSpec((_BLK, D), lambda i: (i, 0)),
            pl.BlockSpec((_BLK, T), lambda i: (i, 0)),
            full((3 * H, D)), full((3 * H, H)), full((1, 3 * H)), full((1, 3 * H)),
            full((3 * H, D)), full((3 * H, H)), full((1, 3 * H)), full((1, 3 * H)),
            full((H, 2 * H)), full((1, H)), full((H, D)), full((1, H)),
            full((128, H)), full((1, 128)),
        ],
        out_specs=[
            pl.BlockSpec((_BLK, 128), lambda i: (i, 0)),
        ],
        out_shape=[
            jax.ShapeDtypeStruct((B, 128), f32),
        ],
    )(xs, lab, x,
      Wi_f, Wh_f, bi_f.reshape(1, -1), bh_f.reshape(1, -1),
      Wi_b, Wh_b, bi_b.reshape(1, -1), bh_b.reshape(1, -1),
      ctx_W, ctx_b.reshape(1, -1), inp_W, inp_b.reshape(1, -1),
      headW, headb)[0]
    return (head[:, :A], head[:, A:A + 1])


# speedup vs baseline: 1.0035x; 1.0035x over previous
import jax
import jax.numpy as jnp
from jax.experimental import pallas as pl
from jax.experimental.pallas import tpu as pltpu
from jax.experimental.pallas import tpu_sc as plsc

_H = 256
_WINDOW = 128
_BLK = 256


def _gather_rows(emb, idx):
    n = idx.shape[0]
    d = emb.shape[1]
    idx2 = idx.reshape(1, n)

    @pl.kernel(
        out_type=jax.ShapeDtypeStruct((n, d), emb.dtype),
        mesh=plsc.VectorSubcoreMesh(core_axis_name="core",
                                    subcore_axis_name="subcore"),
    )
    def gather_kernel(x_hbm, i_hbm, o_hbm):
        def body(i_vmem, o_vmem):
            pltpu.sync_copy(x_hbm.at[i_vmem.at[0]], o_vmem)

        pltpu.emit_pipeline(
            body,
            grid=(n // _WINDOW,),
            in_specs=[pl.BlockSpec((1, _WINDOW), index_map=lambda i: (0, i))],
            out_specs=[pl.BlockSpec((_WINDOW, d), index_map=lambda i: (i, 0))],
            core_axis_name=("core", "subcore"),
            dimension_semantics=(pltpu.PARALLEL,),
        )(i_hbm, o_hbm)

    return gather_kernel(emb, idx2)


def _dense_body(xs_ref, lab_ref, xid_ref,
                Wif_ref, Whf_ref, bif_ref, bhf_ref,
                Wib_ref, Whb_ref, bib_ref, bhb_ref,
                ctxW_ref, ctxb_ref, inpW_ref, inpb_ref,
                headW_ref, headb_ref,
                head_ref):
    T, BLK, D = xs_ref.shape
    H = _H
    f32 = jnp.float32
    bf16 = jnp.bfloat16

    def dn(a, b):
        return jax.lax.dot_general(a.astype(bf16), b.astype(bf16),
                                   (((1,), (1,)), ((), ())),
                                   preferred_element_type=f32)

    xe = xs_ref[...].reshape(T * BLK, D)
    gi_f = (dn(xe, Wif_ref[...]) + bif_ref[...]).astype(bf16).reshape(T, BLK, 3 * H)
    gi_b = (dn(xe, Wib_ref[...]) + bib_ref[...]).astype(bf16).reshape(T, BLK, 3 * H)

    m = (xid_ref[...] != 0).astype(f32)

    Whf = Whf_ref[...]
    Whb = Whb_ref[...]
    bhf = bhf_ref[...]
    bhb = bhb_ref[...]

    def cell(h, gi_t, Wh, bh):
        gh = dn(h, Wh) + bh
        gi32 = gi_t.astype(f32)
        r = jax.nn.sigmoid(gi32[:, :H] + gh[:, :H])
        z = jax.nn.sigmoid(gi32[:, H:2 * H] + gh[:, H:2 * H])
        n = jnp.tanh(gi32[:, 2 * H:] + r * gh[:, 2 * H:])
        return (1.0 - z) * n + z * h

    h_f = jnp.zeros((BLK, H), f32)
    h_b = jnp.zeros((BLK, H), f32)
    acc_f = jnp.zeros((BLK, H), f32)
    acc_b = jnp.zeros((BLK, H), f32)
    for t in range(T):
        tb = T - 1 - t
        h_f = cell(h_f, gi_f[t], Whf, bhf)
        acc_f = acc_f + m[:, t:t + 1] * h_f
        h_b = cell(h_b, gi_b[tb], Whb, bhb)
        acc_b = acc_b + m[:, tb:tb + 1] * h_b

    cnt = jnp.sum(m, axis=1, keepdims=True)
    cf = acc_f / cnt
    cb = acc_b / cnt

    ctxW = ctxW_ref[...]
    feat = (dn(lab_ref[...], inpW_ref[...]) + inpb_ref[...]
            + dn(cf, ctxW[:, :H]) + dn(cb, ctxW[:, H:]) + ctxb_ref[...])

    hv = dn(feat, headW_ref[...]) + headb_ref[...]
    lane = jax.lax.broadcasted_iota(jnp.int32, hv.shape, 1)
    head_ref[...] = jnp.where(lane < 2, jax.nn.sigmoid(hv), hv)


def kernel(x, label, emb, Wi_f, Wh_f, bi_f, bh_f, Wi_b, Wh_b, bi_b, bh_b,
           ctx_W, ctx_b, inp_W, inp_b, att_W, att_b, crit_W, crit_b):
    B, T = x.shape
    D = emb.shape[1]
    H = _H
    A = att_W.shape[0]
    f32 = jnp.float32

    x = x.astype(jnp.int32)
    label = label.astype(jnp.int32)

    idx = jnp.concatenate([x.T.reshape(-1), label.reshape(-1)])
    idx2 = jnp.stack([2 * idx, 2 * idx + 1], axis=1).reshape(-1)
    rows = _gather_rows(emb.reshape(2 * emb.shape[0], D // 2), idx2)
    rows = rows.reshape(-1, D)
    xs = rows[:B * T].reshape(T, B, D)
    lab = rows[B * T:]

    wd = att_W[0:1] - att_W[1:2]
    headW = jnp.concatenate([wd, -wd, crit_W,
                             jnp.zeros((128 - 3, H), f32)], axis=0)
    bd = att_b[0] - att_b[1]
    headb = jnp.concatenate([bd[None], -bd[None], crit_b,
                             jnp.zeros((128 - 3,), f32)]).reshape(1, 128)

    def full(shape):
        return pl.BlockSpec(shape, lambda *_: tuple(0 for _ in shape))

    grid = (B // _BLK,)
    head = pl.pallas_call(
        _dense_body,
        grid=grid,
        in_specs=[
            pl.BlockSpec((T, _BLK, D), lambda i: (0, i, 0)),
            pl.BlockSpec((_BLK, D), lambda i: (i, 0)),
            pl.BlockSpec((_BLK, T), lambda i: (i, 0)),
            full((3 * H, D)), full((3 * H, H)), full((1, 3 * H)), full((1, 3 * H)),
            full((3 * H, D)), full((3 * H, H)), full((1, 3 * H)), full((1, 3 * H)),
            full((H, 2 * H)), full((1, H)), full((H, D)), full((1, H)),
            full((128, H)), full((1, 128)),
        ],
        out_specs=[
            pl.BlockSpec((_BLK, 128), lambda i: (i, 0)),
        ],
        out_shape=[
            jax.ShapeDtypeStruct((B, 128), f32),
        ],
    )(xs, lab, x,
      Wi_f, Wh_f, bi_f.reshape(1, -1), bh_f.reshape(1, -1),
      Wi_b, Wh_b, bi_b.reshape(1, -1), bh_b.reshape(1, -1),
      ctx_W, ctx_b.reshape(1, -1), inp_W, inp_b.reshape(1, -1),
      headW, headb)[0]
    return (head[:, :A], head[:, A:A + 1])


# speedup vs baseline: 13.4166x; 13.3695x over previous
import jax
import jax.numpy as jnp
from jax.experimental import pallas as pl
from jax.experimental.pallas import tpu as pltpu
from jax.experimental.pallas import tpu_sc as plsc

_H = 256
_WINDOW = 64
_BLK = 256


def _gather_rows(emb, idx):
    n = idx.shape[0]
    d = emb.shape[1]
    nw = n // _WINDOW
    idxp = jnp.pad(idx.reshape(nw, _WINDOW),
                   ((0, 0), (0, 128 - _WINDOW))).reshape(1, nw * 128)

    @pl.kernel(
        out_type=jax.ShapeDtypeStruct((n, d), emb.dtype),
        mesh=plsc.VectorSubcoreMesh(core_axis_name="core",
                                    subcore_axis_name="subcore"),
    )
    def gather_kernel(x_hbm, i_hbm, o_hbm):
        def body(i_vmem, o_vmem):
            pltpu.sync_copy(x_hbm.at[i_vmem.at[0, pl.ds(0, _WINDOW)]], o_vmem)

        pltpu.emit_pipeline(
            body,
            grid=(nw,),
            in_specs=[pl.BlockSpec((1, 128), index_map=lambda i: (0, i))],
            out_specs=[pl.BlockSpec((_WINDOW, d), index_map=lambda i: (i, 0))],
            core_axis_name=("core", "subcore"),
            dimension_semantics=(pltpu.PARALLEL,),
        )(i_hbm, o_hbm)

    return gather_kernel(emb, idxp)


def _dense_body(xs_ref, lab_ref, xid_ref,
                Wif_ref, Whf_ref, bif_ref, bhf_ref,
                Wib_ref, Whb_ref, bib_ref, bhb_ref,
                ctxW_ref, ctxb_ref, inpW_ref, inpb_ref,
                headW_ref, headb_ref,
                head_ref):
    T, BLK, D = xs_ref.shape
    H = _H
    f32 = jnp.float32
    bf16 = jnp.bfloat16

    def dn(a, b):
        return jax.lax.dot_general(a.astype(bf16), b.astype(bf16),
                                   (((1,), (1,)), ((), ())),
                                   preferred_element_type=f32)

    xe = xs_ref[...].reshape(T * BLK, D)
    gi_f = (dn(xe, Wif_ref[...]) + bif_ref[...]).astype(bf16).reshape(T, BLK, 3 * H)
    gi_b = (dn(xe, Wib_ref[...]) + bib_ref[...]).astype(bf16).reshape(T, BLK, 3 * H)

    m = (xid_ref[...] != 0).astype(f32)

    Whf = Whf_ref[...]
    Whb = Whb_ref[...]
    bhf = bhf_ref[...]
    bhb = bhb_ref[...]

    def cell(h, gi_t, Wh, bh):
        gh = dn(h, Wh) + bh
        gi32 = gi_t.astype(f32)
        r = jax.nn.sigmoid(gi32[:, :H] + gh[:, :H])
        z = jax.nn.sigmoid(gi32[:, H:2 * H] + gh[:, H:2 * H])
        n = jnp.tanh(gi32[:, 2 * H:] + r * gh[:, 2 * H:])
        return (1.0 - z) * n + z * h

    h_f = jnp.zeros((BLK, H), f32)
    h_b = jnp.zeros((BLK, H), f32)
    acc_f = jnp.zeros((BLK, H), f32)
    acc_b = jnp.zeros((BLK, H), f32)
    for t in range(T):
        tb = T - 1 - t
        h_f = cell(h_f, gi_f[t], Whf, bhf)
        acc_f = acc_f + m[:, t:t + 1] * h_f
        h_b = cell(h_b, gi_b[tb], Whb, bhb)
        acc_b = acc_b + m[:, tb:tb + 1] * h_b

    cnt = jnp.sum(m, axis=1, keepdims=True)
    cf = acc_f / cnt
    cb = acc_b / cnt

    ctxW = ctxW_ref[...]
    feat = (dn(lab_ref[...], inpW_ref[...]) + inpb_ref[...]
            + dn(cf, ctxW[:, :H]) + dn(cb, ctxW[:, H:]) + ctxb_ref[...])

    hv = dn(feat, headW_ref[...]) + headb_ref[...]
    lane = jax.lax.broadcasted_iota(jnp.int32, hv.shape, 1)
    head_ref[...] = jnp.where(lane < 2, jax.nn.sigmoid(hv), hv)


def kernel(x, label, emb, Wi_f, Wh_f, bi_f, bh_f, Wi_b, Wh_b, bi_b, bh_b,
           ctx_W, ctx_b, inp_W, inp_b, att_W, att_b, crit_W, crit_b):
    B, T = x.shape
    D = emb.shape[1]
    H = _H
    A = att_W.shape[0]
    f32 = jnp.float32

    x = x.astype(jnp.int32)
    label = label.astype(jnp.int32)

    idx = jnp.concatenate([x.T.reshape(-1), label.reshape(-1)])
    rows = _gather_rows(emb, idx)
    xs = rows[:B * T].reshape(T, B, D)
    lab = rows[B * T:]

    wd = att_W[0:1] - att_W[1:2]
    headW = jnp.concatenate([wd, -wd, crit_W,
                             jnp.zeros((128 - 3, H), f32)], axis=0)
    bd = att_b[0] - att_b[1]
    headb = jnp.concatenate([bd[None], -bd[None], crit_b,
                             jnp.zeros((128 - 3,), f32)]).reshape(1, 128)

    def full(shape):
        return pl.BlockSpec(shape, lambda *_: tuple(0 for _ in shape))

    grid = (B // _BLK,)
    head = pl.pallas_call(
        _dense_body,
        grid=grid,
        in_specs=[
            pl.BlockSpec((T, _BLK, D), lambda i: (0, i, 0)),
            pl.BlockSpec((_BLK, D), lambda i: (i, 0)),
            pl.BlockSpec((_BLK, T), lambda i: (i, 0)),
            full((3 * H, D)), full((3 * H, H)), full((1, 3 * H)), full((1, 3 * H)),
            full((3 * H, D)), full((3 * H, H)), full((1, 3 * H)), full((1, 3 * H)),
            full((H, 2 * H)), full((1, H)), full((H, D)), full((1, H)),
            full((128, H)), full((1, 128)),
        ],
        out_specs=[
            pl.BlockSpec((_BLK, 128), lambda i: (i, 0)),
        ],
        out_shape=[
            jax.ShapeDtypeStruct((B, 128), f32),
        ],
    )(xs, lab, x,
      Wi_f, Wh_f, bi_f.reshape(1, -1), bh_f.reshape(1, -1),
      Wi_b, Wh_b, bi_b.reshape(1, -1), bh_b.reshape(1, -1),
      ctx_W, ctx_b.reshape(1, -1), inp_W, inp_b.reshape(1, -1),
      headW, headb)[0]
    return (head[:, :A], head[:, A:A + 1])


# speedup vs baseline: 15.6564x; 1.1669x over previous
import jax
import jax.numpy as jnp
from jax.experimental import pallas as pl
from jax.experimental.pallas import tpu as pltpu
from jax.experimental.pallas import tpu_sc as plsc

_H = 256
_WINDOW = 64
_BLK = 256


def _gather_rows(emb, idx):
    n = idx.shape[0]
    d = emb.shape[1]
    nw = n // _WINDOW
    idxp = jnp.pad(idx.reshape(nw, _WINDOW),
                   ((0, 0), (0, 128 - _WINDOW))).reshape(1, nw * 128)

    @pl.kernel(
        out_type=jax.ShapeDtypeStruct((n, d), emb.dtype),
        mesh=plsc.VectorSubcoreMesh(core_axis_name="core",
                                    subcore_axis_name="subcore"),
    )
    def gather_kernel(x_hbm, i_hbm, o_hbm):
        def body(i_vmem, o_vmem):
            pltpu.sync_copy(x_hbm.at[i_vmem.at[0, pl.ds(0, _WINDOW)]], o_vmem)

        pltpu.emit_pipeline(
            body,
            grid=(nw,),
            in_specs=[pl.BlockSpec((1, 128), index_map=lambda i: (0, i))],
            out_specs=[pl.BlockSpec((_WINDOW, d), index_map=lambda i: (i, 0))],
            core_axis_name=("core", "subcore"),
            dimension_semantics=(pltpu.PARALLEL,),
        )(i_hbm, o_hbm)

    return gather_kernel(emb, idxp)


def _dense_body(xs_ref, lab_ref, xid_ref,
                Wif_ref, Whf_ref, bif_ref, bhf_ref,
                Wib_ref, Whb_ref, bib_ref, bhb_ref,
                ctxW_ref, ctxb_ref, inpW_ref, inpb_ref,
                headW_ref, headb_ref,
                head_ref):
    T, BLK, D = xs_ref.shape
    H = _H
    f32 = jnp.float32
    bf16 = jnp.bfloat16

    def dn(a, b):
        return jax.lax.dot_general(a.astype(bf16), b.astype(bf16),
                                   (((1,), (1,)), ((), ())),
                                   preferred_element_type=f32)

    xe = xs_ref[...].reshape(T * BLK, D)
    gi_f = (dn(xe, Wif_ref[...]) + bif_ref[...]).astype(bf16).reshape(T, BLK, 3 * H)
    gi_b = (dn(xe, Wib_ref[...]) + bib_ref[...]).astype(bf16).reshape(T, BLK, 3 * H)

    m = (xid_ref[...] != 0).astype(f32)

    Whf = Whf_ref[...]
    Whb = Whb_ref[...]
    bhf = bhf_ref[...]
    bhb = bhb_ref[...]

    def cell(h, gi_t, Wh, bh):
        gh = dn(h, Wh) + bh
        gi32 = gi_t.astype(f32)
        r = jax.nn.sigmoid(gi32[:, :H] + gh[:, :H])
        z = jax.nn.sigmoid(gi32[:, H:2 * H] + gh[:, H:2 * H])
        n = jnp.tanh(gi32[:, 2 * H:] + r * gh[:, 2 * H:])
        return (1.0 - z) * n + z * h

    h_f = jnp.zeros((BLK, H), f32)
    h_b = jnp.zeros((BLK, H), f32)
    acc_f = jnp.zeros((BLK, H), f32)
    acc_b = jnp.zeros((BLK, H), f32)
    for t in range(T):
        tb = T - 1 - t
        h_f = cell(h_f, gi_f[t], Whf, bhf)
        acc_f = acc_f + m[:, t:t + 1] * h_f
        h_b = cell(h_b, gi_b[tb], Whb, bhb)
        acc_b = acc_b + m[:, tb:tb + 1] * h_b

    cnt = jnp.sum(m, axis=1, keepdims=True)
    cf = acc_f / cnt
    cb = acc_b / cnt

    ctxW = ctxW_ref[...]
    feat = (dn(lab_ref[...], inpW_ref[...]) + inpb_ref[...]
            + dn(cf, ctxW[:, :H]) + dn(cb, ctxW[:, H:]) + ctxb_ref[...])

    hv = dn(feat, headW_ref[...]) + headb_ref[...]
    lane = jax.lax.broadcasted_iota(jnp.int32, hv.shape, 1)
    head_ref[...] = jnp.where(lane < 2, jax.nn.sigmoid(hv), hv)


def kernel(x, label, emb, Wi_f, Wh_f, bi_f, bh_f, Wi_b, Wh_b, bi_b, bh_b,
           ctx_W, ctx_b, inp_W, inp_b, att_W, att_b, crit_W, crit_b):
    B, T = x.shape
    D = emb.shape[1]
    H = _H
    A = att_W.shape[0]
    f32 = jnp.float32

    x = x.astype(jnp.int32)
    label = label.astype(jnp.int32)

    xs = _gather_rows(emb, x.T.reshape(-1)).reshape(T, B, D)
    lab = _gather_rows(emb, label.reshape(-1))

    wd = att_W[0:1] - att_W[1:2]
    headW = jnp.concatenate([wd, -wd, crit_W,
                             jnp.zeros((128 - 3, H), f32)], axis=0)
    bd = att_b[0] - att_b[1]
    headb = jnp.concatenate([bd[None], -bd[None], crit_b,
                             jnp.zeros((128 - 3,), f32)]).reshape(1, 128)

    def full(shape):
        return pl.BlockSpec(shape, lambda *_: tuple(0 for _ in shape))

    grid = (B // _BLK,)
    head = pl.pallas_call(
        _dense_body,
        grid=grid,
        in_specs=[
            pl.BlockSpec((T, _BLK, D), lambda i: (0, i, 0)),
            pl.BlockSpec((_BLK, D), lambda i: (i, 0)),
            pl.BlockSpec((_BLK, T), lambda i: (i, 0)),
            full((3 * H, D)), full((3 * H, H)), full((1, 3 * H)), full((1, 3 * H)),
            full((3 * H, D)), full((3 * H, H)), full((1, 3 * H)), full((1, 3 * H)),
            full((H, 2 * H)), full((1, H)), full((H, D)), full((1, H)),
            full((128, H)), full((1, 128)),
        ],
        out_specs=[
            pl.BlockSpec((_BLK, 128), lambda i: (i, 0)),
        ],
        out_shape=[
            jax.ShapeDtypeStruct((B, 128), f32),
        ],
    )(xs, lab, x,
      Wi_f, Wh_f, bi_f.reshape(1, -1), bh_f.reshape(1, -1),
      Wi_b, Wh_b, bi_b.reshape(1, -1), bh_b.reshape(1, -1),
      ctx_W, ctx_b.reshape(1, -1), inp_W, inp_b.reshape(1, -1),
      headW, headb)[0]
    return (head[:, :A], head[:, A:A + 1])
